# baseline (speedup 1.0000x reference)
import functools

import jax
import jax.numpy as jnp
from jax import lax
from jax.experimental import pallas as pl
from jax.experimental.pallas import tpu as pltpu
from jax.experimental.pallas import tpu_sc as plsc

N_NODES = 10000
N_EDGES = 320000
D = 128
LANES = 16

NUM_CORES = 2
NUM_SUBCORES = 16
NUM_WORKERS = NUM_CORES * NUM_SUBCORES
EDGES_PER_WORKER = N_EDGES // NUM_WORKERS
CHUNK = 80
NUM_CHUNKS = EDGES_PER_WORKER // CHUNK
STRIPE = 624
TAIL = N_NODES - NUM_SUBCORES * STRIPE
ZROWS = 48

_SDS = jax.ShapeDtypeStruct


def _fill_zero(buf, nrows):
  zero16 = jnp.zeros((LANES,), jnp.float32)

  def body(k, carry):
    buf[k // (D // LANES), pl.ds((k % (D // LANES)) * LANES, LANES)] = zero16
    return carry
  lax.fori_loop(0, nrows * (D // LANES), body, 0)


def _zero_and_writeout_specs(s):
  return s * STRIPE


def _make_row_agg():
  mesh = plsc.VectorSubcoreMesh(
      core_axis_name="c", subcore_axis_name="s",
      num_cores=NUM_CORES, num_subcores=NUM_SUBCORES)

  @functools.partial(
      pl.kernel, mesh=mesh,
      out_type=_SDS((NUM_CORES, N_NODES, D), jnp.float32),
      scratch_types=(
          pltpu.VMEM_SHARED((N_NODES, D), jnp.float32),
          pltpu.VMEM((CHUNK,), jnp.int32),
          pltpu.VMEM((CHUNK,), jnp.int32),
          pltpu.VMEM((CHUNK,), jnp.int32),
          pltpu.VMEM((CHUNK,), jnp.int32),
          pltpu.VMEM((CHUNK, D), jnp.float32),
          pltpu.VMEM((CHUNK, D), jnp.float32),
          pltpu.VMEM((ZROWS, D), jnp.float32),
          pltpu.SemaphoreType.DMA,
          pltpu.SemaphoreType.DMA,
          pltpu.SemaphoreType.DMA,
          pltpu.SemaphoreType.DMA,
      ))
  def agg(x_hbm, src_hbm, dst_hbm, part_hbm, acc,
          sidx0, sidx1, didx0, didx1, rows0, rows1, zbuf,
          gsem0, gsem1, ssem0, ssem1):
    c = lax.axis_index("c")
    s = lax.axis_index("s")
    wid = s * NUM_CORES + c

    _fill_zero(zbuf, ZROWS)
    off = s * STRIPE
    for r in range(STRIPE // ZROWS):
      pltpu.sync_copy(zbuf, acc.at[pl.ds(off + r * ZROWS, ZROWS)])

    @pl.when(s == NUM_SUBCORES - 1)
    def _zero_tail():
      pltpu.sync_copy(zbuf.at[pl.ds(0, TAIL)],
                      acc.at[pl.ds(NUM_SUBCORES * STRIPE, TAIL)])
    plsc.subcore_barrier()

    base_w = wid * EDGES_PER_WORKER

    def load_idx(sb, db, base):
      pltpu.sync_copy(src_hbm.at[pl.ds(base, CHUNK)], sb)
      pltpu.sync_copy(dst_hbm.at[pl.ds(base, CHUNK)], db)

    def wait_g(sb, rb, sem):
      pltpu.make_async_copy(x_hbm.at[sb], rb, sem).wait()

    def wait_s(rb, db, sem):
      pltpu.make_async_copy(rb, acc.at[db], sem).wait()

    load_idx(sidx0, didx0, base_w)
    pltpu.async_copy(x_hbm.at[sidx0], rows0, gsem0)
    load_idx(sidx1, didx1, base_w + CHUNK)
    pltpu.async_copy(x_hbm.at[sidx1], rows1, gsem1)
    wait_g(sidx0, rows0, gsem0)
    pltpu.async_copy(rows0, acc.at[didx0], ssem0, add=True)
    wait_g(sidx1, rows1, gsem1)
    pltpu.async_copy(rows1, acc.at[didx1], ssem1, add=True)
    wait_s(rows0, didx0, ssem0)
    load_idx(sidx0, didx0, base_w + 2 * CHUNK)
    pltpu.async_copy(x_hbm.at[sidx0], rows0, gsem0)

    def pair(k, carry):
      wait_s(rows1, didx1, ssem1)
      load_idx(sidx1, didx1, base_w + (2 * k + 1) * CHUNK)
      pltpu.async_copy(x_hbm.at[sidx1], rows1, gsem1)
      wait_g(sidx0, rows0, gsem0)
      pltpu.async_copy(rows0, acc.at[didx0], ssem0, add=True)
      wait_g(sidx1, rows1, gsem1)
      pltpu.async_copy(rows1, acc.at[didx1], ssem1, add=True)
      wait_s(rows0, didx0, ssem0)
      load_idx(sidx0, didx0, base_w + (2 * k + 2) * CHUNK)
      pltpu.async_copy(x_hbm.at[sidx0], rows0, gsem0)
      return carry
    lax.fori_loop(1, (NUM_CHUNKS - 1) // 2, pair, 0)
    wait_s(rows1, didx1, ssem1)
    wait_g(sidx0, rows0, gsem0)
    pltpu.async_copy(rows0, acc.at[didx0], ssem0, add=True)
    wait_s(rows0, didx0, ssem0)

    plsc.subcore_barrier()
    pltpu.sync_copy(acc.at[pl.ds(off, STRIPE)],
                    part_hbm.at[c, pl.ds(off, STRIPE)])

    @pl.when(s == NUM_SUBCORES - 1)
    def _write_tail():
      toff = NUM_SUBCORES * STRIPE
      pltpu.sync_copy(acc.at[pl.ds(toff, TAIL)],
                      part_hbm.at[c, pl.ds(toff, TAIL)])

  return agg


def _make_deg():
  mesh = plsc.VectorSubcoreMesh(
      core_axis_name="c", subcore_axis_name="s",
      num_cores=NUM_CORES, num_subcores=NUM_SUBCORES)

  @functools.partial(
      pl.kernel, mesh=mesh,
      out_type=_SDS((NUM_CORES, N_NODES, D), jnp.float32),
      scratch_types=(
          pltpu.VMEM_SHARED((N_NODES, D), jnp.float32),
          pltpu.VMEM((CHUNK,), jnp.int32),
          pltpu.VMEM((CHUNK,), jnp.int32),
          pltpu.VMEM((CHUNK, D), jnp.float32),
          pltpu.VMEM((ZROWS, D), jnp.float32),
          pltpu.SemaphoreType.DMA,
          pltpu.SemaphoreType.DMA,
      ))
  def deg(dst_hbm, deg_hbm, acc, didx0, didx1, ones, zbuf, sem0, sem1):
    c = lax.axis_index("c")
    s = lax.axis_index("s")
    wid = s * NUM_CORES + c

    _fill_zero(zbuf, ZROWS)
    one16 = jnp.full((LANES,), 1.0, jnp.float32)

    def ofill(k, carry):
      ones[k // (D // LANES), pl.ds((k % (D // LANES)) * LANES, LANES)] = one16
      return carry
    lax.fori_loop(0, CHUNK * (D // LANES), ofill, 0)

    off = s * STRIPE
    for r in range(STRIPE // ZROWS):
      pltpu.sync_copy(zbuf, acc.at[pl.ds(off + r * ZROWS, ZROWS)])

    @pl.when(s == NUM_SUBCORES - 1)
    def _zero_tail():
      pltpu.sync_copy(zbuf.at[pl.ds(0, TAIL)],
                      acc.at[pl.ds(NUM_SUBCORES * STRIPE, TAIL)])
    plsc.subcore_barrier()

    base_w = wid * EDGES_PER_WORKER

    pltpu.sync_copy(dst_hbm.at[pl.ds(base_w, CHUNK)], didx0)
    pltpu.async_copy(ones, acc.at[didx0], sem0, add=True)

    def pair(k, carry):
      pltpu.sync_copy(dst_hbm.at[pl.ds(base_w + (2 * k + 1) * CHUNK, CHUNK)],
                      didx1)
      pltpu.async_copy(ones, acc.at[didx1], sem1, add=True)
      pltpu.make_async_copy(ones, acc.at[didx0], sem0).wait()
      pltpu.sync_copy(dst_hbm.at[pl.ds(base_w + (2 * k + 2) * CHUNK, CHUNK)],
                      didx0)
      pltpu.async_copy(ones, acc.at[didx0], sem0, add=True)
      pltpu.make_async_copy(ones, acc.at[didx1], sem1).wait()
      return carry
    lax.fori_loop(0, (NUM_CHUNKS - 1) // 2, pair, 0)
    pltpu.make_async_copy(ones, acc.at[didx0], sem0).wait()

    plsc.subcore_barrier()
    pltpu.sync_copy(acc.at[pl.ds(off, STRIPE)],
                    deg_hbm.at[c, pl.ds(off, STRIPE)])

    @pl.when(s == NUM_SUBCORES - 1)
    def _write_tail():
      toff = NUM_SUBCORES * STRIPE
      pltpu.sync_copy(acc.at[pl.ds(toff, TAIL)],
                      deg_hbm.at[c, pl.ds(toff, TAIL)])

  return deg


def _make_tc_combine(relu, block_rows=1000):
  def body(x_ref, p_ref, dg_ref, ws_ref, wn_ref, b_ref, o_ref):
    agg = p_ref[0] + p_ref[1]
    deg = dg_ref[0] + dg_ref[1]
    inv = 1.0 / jnp.maximum(jnp.max(deg, axis=1, keepdims=True), 1.0)
    h = jnp.dot(x_ref[...], ws_ref[...], preferred_element_type=jnp.float32)
    h = h + jnp.dot(agg * inv, wn_ref[...],
                    preferred_element_type=jnp.float32)
    h = h + b_ref[...]
    if relu:
      h = jnp.maximum(h, 0.0)
    o_ref[...] = h

  return pl.pallas_call(
      body,
      grid=(N_NODES // block_rows,),
      in_specs=[
          pl.BlockSpec((block_rows, D), lambda i: (i, 0)),
          pl.BlockSpec((NUM_CORES, block_rows, D), lambda i: (0, i, 0)),
          pl.BlockSpec((NUM_CORES, block_rows, D), lambda i: (0, i, 0)),
          pl.BlockSpec((D, D), lambda i: (0, 0)),
          pl.BlockSpec((D, D), lambda i: (0, 0)),
          pl.BlockSpec((1, D), lambda i: (0, 0)),
      ],
      out_specs=pl.BlockSpec((block_rows, D), lambda i: (i, 0)),
      out_shape=_SDS((N_NODES, D), jnp.float32),
  )


_get_row_agg = functools.lru_cache(maxsize=None)(_make_row_agg)
_get_deg = functools.lru_cache(maxsize=None)(_make_deg)
_combine_relu = _make_tc_combine(relu=True)
_combine_linear = _make_tc_combine(relu=False)


def kernel(in_feat, edge_index, W1_self, W1_neigh, b1, W2_self, W2_neigh, b2):
  src = edge_index[0].astype(jnp.int32)
  dst = edge_index[1].astype(jnp.int32)
  degp = _get_deg()(dst)
  part1 = _get_row_agg()(in_feat, src, dst)
  h1 = _combine_relu(in_feat, part1, degp, W1_self, W1_neigh,
                     b1.reshape(1, D))
  part2 = _get_row_agg()(h1, src, dst)
  out = _combine_linear(h1, part2, degp, W2_self, W2_neigh,
                        b2.reshape(1, D))
  return out


# speedup vs baseline: 1.1509x; 1.1509x over previous
import functools

import jax
import jax.numpy as jnp
from jax import lax
from jax.experimental import pallas as pl
from jax.experimental.pallas import tpu as pltpu
from jax.experimental.pallas import tpu_sc as plsc

N_NODES = 10000
N_EDGES = 320000
D = 128
LANES = 16

NUM_CORES = 2
NUM_SUBCORES = 16
NUM_WORKERS = NUM_CORES * NUM_SUBCORES
EDGES_PER_WORKER = N_EDGES // NUM_WORKERS
CHUNK = 80
NUM_CHUNKS = EDGES_PER_WORKER // CHUNK
STRIPE = 624
TAIL = N_NODES - NUM_SUBCORES * STRIPE
ZROWS = 48

_SDS = jax.ShapeDtypeStruct


def _fill_zero(buf, nrows):
  zero16 = jnp.zeros((LANES,), jnp.float32)

  def body(k, carry):
    buf[k // (D // LANES), pl.ds((k % (D // LANES)) * LANES, LANES)] = zero16
    return carry
  lax.fori_loop(0, nrows * (D // LANES), body, 0)


def _zero_and_writeout_specs(s):
  return s * STRIPE


def _make_row_agg():
  mesh = plsc.VectorSubcoreMesh(
      core_axis_name="c", subcore_axis_name="s",
      num_cores=NUM_CORES, num_subcores=NUM_SUBCORES)

  @functools.partial(
      pl.kernel, mesh=mesh,
      out_type=_SDS((NUM_CORES, N_NODES, D), jnp.float32),
      scratch_types=(
          pltpu.VMEM_SHARED((N_NODES, D), jnp.float32),
          pltpu.VMEM((2 * CHUNK,), jnp.int32),
          pltpu.VMEM((2 * CHUNK,), jnp.int32),
          pltpu.VMEM((2 * CHUNK,), jnp.int32),
          pltpu.VMEM((2 * CHUNK,), jnp.int32),
          pltpu.VMEM((CHUNK,), jnp.int32),
          pltpu.VMEM((CHUNK,), jnp.int32),
          pltpu.VMEM((CHUNK, D), jnp.float32),
          pltpu.VMEM((CHUNK, D), jnp.float32),
          pltpu.VMEM((ZROWS, D), jnp.float32),
          pltpu.SemaphoreType.DMA,
          pltpu.SemaphoreType.DMA,
          pltpu.SemaphoreType.DMA,
          pltpu.SemaphoreType.DMA,
      ))
  def agg(x_hbm, src_hbm, dst_hbm, part_hbm, acc,
          srcA, srcB, dstA, dstB, didx0, didx1, rows0, rows1, zbuf,
          gsem0, gsem1, isemA, isemB):
    c = lax.axis_index("c")
    s = lax.axis_index("s")
    wid = s * NUM_CORES + c

    _fill_zero(zbuf, ZROWS)
    off = s * STRIPE
    for r in range(STRIPE // ZROWS):
      pltpu.sync_copy(zbuf, acc.at[pl.ds(off + r * ZROWS, ZROWS)])

    @pl.when(s == NUM_SUBCORES - 1)
    def _zero_tail():
      pltpu.sync_copy(zbuf.at[pl.ds(0, TAIL)],
                      acc.at[pl.ds(NUM_SUBCORES * STRIPE, TAIL)])
    plsc.subcore_barrier()

    base_w = wid * EDGES_PER_WORKER

    def start_pair_load(sb, db, sem, p):
      base = base_w + p * 2 * CHUNK
      pltpu.async_copy(src_hbm.at[pl.ds(base, 2 * CHUNK)], sb, sem)
      pltpu.async_copy(dst_hbm.at[pl.ds(base, 2 * CHUNK)], db, sem)

    def wait_pair_load(sb, db, sem, p):
      base = base_w + p * 2 * CHUNK
      pltpu.make_async_copy(src_hbm.at[pl.ds(base, 2 * CHUNK)], sb, sem).wait()
      pltpu.make_async_copy(dst_hbm.at[pl.ds(base, 2 * CHUNK)], db, sem).wait()

    def copy_didx(db_pair, off, db):
      for j in range(CHUNK // LANES):
        db[pl.ds(j * LANES, LANES)] = db_pair[pl.ds(off + j * LANES, LANES)]

    def run_pair(sb, db):
      pltpu.async_copy(x_hbm.at[sb.at[pl.ds(0, CHUNK)]], rows0, gsem0)
      copy_didx(db, 0, didx0)
      pltpu.async_copy(x_hbm.at[sb.at[pl.ds(CHUNK, CHUNK)]], rows1, gsem1)
      copy_didx(db, CHUNK, didx1)
      pltpu.make_async_copy(x_hbm.at[sb.at[pl.ds(0, CHUNK)]],
                            rows0, gsem0).wait()
      pltpu.sync_copy(rows0, acc.at[didx0], add=True)
      pltpu.make_async_copy(x_hbm.at[sb.at[pl.ds(CHUNK, CHUNK)]],
                            rows1, gsem1).wait()
      pltpu.sync_copy(rows1, acc.at[didx1], add=True)

    NPAIRS = NUM_CHUNKS // 2
    start_pair_load(srcA, dstA, isemA, 0)
    start_pair_load(srcB, dstB, isemB, 1)

    def quad(q, carry):
      wait_pair_load(srcA, dstA, isemA, 2 * q)
      run_pair(srcA, dstA)

      @pl.when(q < NPAIRS // 2 - 1)
      def _():
        start_pair_load(srcA, dstA, isemA, 2 * q + 2)
      wait_pair_load(srcB, dstB, isemB, 2 * q + 1)
      run_pair(srcB, dstB)

      @pl.when(q < NPAIRS // 2 - 1)
      def _():
        start_pair_load(srcB, dstB, isemB, 2 * q + 3)
      return carry
    lax.fori_loop(0, NPAIRS // 2, quad, 0)

    tb = base_w + (NUM_CHUNKS - 1) * CHUNK
    pltpu.sync_copy(src_hbm.at[pl.ds(tb, CHUNK)], srcA.at[pl.ds(0, CHUNK)])
    pltpu.sync_copy(dst_hbm.at[pl.ds(tb, CHUNK)], didx0)
    pltpu.async_copy(x_hbm.at[srcA.at[pl.ds(0, CHUNK)]], rows0, gsem0)
    pltpu.make_async_copy(x_hbm.at[srcA.at[pl.ds(0, CHUNK)]],
                          rows0, gsem0).wait()
    pltpu.sync_copy(rows0, acc.at[didx0], add=True)

    plsc.subcore_barrier()
    pltpu.sync_copy(acc.at[pl.ds(off, STRIPE)],
                    part_hbm.at[c, pl.ds(off, STRIPE)])

    @pl.when(s == NUM_SUBCORES - 1)
    def _write_tail():
      toff = NUM_SUBCORES * STRIPE
      pltpu.sync_copy(acc.at[pl.ds(toff, TAIL)],
                      part_hbm.at[c, pl.ds(toff, TAIL)])

  return agg


def _make_deg():
  mesh = plsc.VectorSubcoreMesh(
      core_axis_name="c", subcore_axis_name="s",
      num_cores=NUM_CORES, num_subcores=NUM_SUBCORES)

  @functools.partial(
      pl.kernel, mesh=mesh,
      out_type=_SDS((NUM_CORES, N_NODES, D), jnp.float32),
      scratch_types=(
          pltpu.VMEM_SHARED((N_NODES, D), jnp.float32),
          pltpu.VMEM((CHUNK,), jnp.int32),
          pltpu.VMEM((CHUNK,), jnp.int32),
          pltpu.VMEM((CHUNK, D), jnp.float32),
          pltpu.VMEM((ZROWS, D), jnp.float32),
          pltpu.SemaphoreType.DMA,
          pltpu.SemaphoreType.DMA,
      ))
  def deg(dst_hbm, deg_hbm, acc, didx0, didx1, ones, zbuf, sem0, sem1):
    c = lax.axis_index("c")
    s = lax.axis_index("s")
    wid = s * NUM_CORES + c

    _fill_zero(zbuf, ZROWS)
    one16 = jnp.full((LANES,), 1.0, jnp.float32)

    def ofill(k, carry):
      ones[k // (D // LANES), pl.ds((k % (D // LANES)) * LANES, LANES)] = one16
      return carry
    lax.fori_loop(0, CHUNK * (D // LANES), ofill, 0)

    off = s * STRIPE
    for r in range(STRIPE // ZROWS):
      pltpu.sync_copy(zbuf, acc.at[pl.ds(off + r * ZROWS, ZROWS)])

    @pl.when(s == NUM_SUBCORES - 1)
    def _zero_tail():
      pltpu.sync_copy(zbuf.at[pl.ds(0, TAIL)],
                      acc.at[pl.ds(NUM_SUBCORES * STRIPE, TAIL)])
    plsc.subcore_barrier()

    base_w = wid * EDGES_PER_WORKER

    pltpu.sync_copy(dst_hbm.at[pl.ds(base_w, CHUNK)], didx0)
    pltpu.async_copy(ones, acc.at[didx0], sem0, add=True)

    def pair(k, carry):
      pltpu.sync_copy(dst_hbm.at[pl.ds(base_w + (2 * k + 1) * CHUNK, CHUNK)],
                      didx1)
      pltpu.async_copy(ones, acc.at[didx1], sem1, add=True)
      pltpu.make_async_copy(ones, acc.at[didx0], sem0).wait()
      pltpu.sync_copy(dst_hbm.at[pl.ds(base_w + (2 * k + 2) * CHUNK, CHUNK)],
                      didx0)
      pltpu.async_copy(ones, acc.at[didx0], sem0, add=True)
      pltpu.make_async_copy(ones, acc.at[didx1], sem1).wait()
      return carry
    lax.fori_loop(0, (NUM_CHUNKS - 1) // 2, pair, 0)
    pltpu.make_async_copy(ones, acc.at[didx0], sem0).wait()

    plsc.subcore_barrier()
    pltpu.sync_copy(acc.at[pl.ds(off, STRIPE)],
                    deg_hbm.at[c, pl.ds(off, STRIPE)])

    @pl.when(s == NUM_SUBCORES - 1)
    def _write_tail():
      toff = NUM_SUBCORES * STRIPE
      pltpu.sync_copy(acc.at[pl.ds(toff, TAIL)],
                      deg_hbm.at[c, pl.ds(toff, TAIL)])

  return deg


def _make_tc_combine(relu, block_rows=1000):
  def body(x_ref, p_ref, dg_ref, ws_ref, wn_ref, b_ref, o_ref):
    agg = p_ref[0] + p_ref[1]
    deg = dg_ref[0] + dg_ref[1]
    inv = 1.0 / jnp.maximum(jnp.max(deg, axis=1, keepdims=True), 1.0)
    h = jnp.dot(x_ref[...], ws_ref[...], preferred_element_type=jnp.float32)
    h = h + jnp.dot(agg * inv, wn_ref[...],
                    preferred_element_type=jnp.float32)
    h = h + b_ref[...]
    if relu:
      h = jnp.maximum(h, 0.0)
    o_ref[...] = h

  return pl.pallas_call(
      body,
      grid=(N_NODES // block_rows,),
      in_specs=[
          pl.BlockSpec((block_rows, D), lambda i: (i, 0)),
          pl.BlockSpec((NUM_CORES, block_rows, D), lambda i: (0, i, 0)),
          pl.BlockSpec((NUM_CORES, block_rows, D), lambda i: (0, i, 0)),
          pl.BlockSpec((D, D), lambda i: (0, 0)),
          pl.BlockSpec((D, D), lambda i: (0, 0)),
          pl.BlockSpec((1, D), lambda i: (0, 0)),
      ],
      out_specs=pl.BlockSpec((block_rows, D), lambda i: (i, 0)),
      out_shape=_SDS((N_NODES, D), jnp.float32),
  )


_get_row_agg = functools.lru_cache(maxsize=None)(_make_row_agg)
_get_deg = functools.lru_cache(maxsize=None)(_make_deg)
_combine_relu = _make_tc_combine(relu=True)
_combine_linear = _make_tc_combine(relu=False)


def kernel(in_feat, edge_index, W1_self, W1_neigh, b1, W2_self, W2_neigh, b2):
  src = edge_index[0].astype(jnp.int32)
  dst = edge_index[1].astype(jnp.int32)
  degp = _get_deg()(dst)
  part1 = _get_row_agg()(in_feat, src, dst)
  h1 = _combine_relu(in_feat, part1, degp, W1_self, W1_neigh,
                     b1.reshape(1, D))
  part2 = _get_row_agg()(h1, src, dst)
  out = _combine_linear(h1, part2, degp, W2_self, W2_neigh,
                        b2.reshape(1, D))
  return out
